# baseline (speedup 1.0000x reference)
import functools

import jax
import jax.numpy as jnp
from jax import lax
from jax.experimental import pallas as pl
from jax.experimental.pallas import tpu as pltpu
from jax.experimental.pallas import tpu_sc as plsc

N = 10000
E = 320000
F_IN = 128
H = 64
G = 64

NC, NS = 2, 16
NW = NC * NS
CH = 128
NCHUNK = 80
EPW = NCHUNK * CH
E_PAD = NW * EPW
NACC = 10240
RPT = NACC // NS
DW = 16

BK = 240
NPAD = N + BK

_SC_MESH = plsc.VectorSubcoreMesh(core_axis_name="c", subcore_axis_name="s")


def _sc_agg(hs, srcr, dstr, zrows):
    @functools.partial(
        pl.kernel,
        out_type=jax.ShapeDtypeStruct((NC, NACC, H), jnp.float32),
        mesh=_SC_MESH,
        compiler_params=pltpu.CompilerParams(use_tc_tiling_on_sc=False),
        scratch_types=[
            pltpu.VMEM((NCHUNK, CH), jnp.int32),
            pltpu.VMEM((NCHUNK, CH), jnp.int32),
            pltpu.VMEM((CH, H), jnp.float32),
            pltpu.VMEM((CH, H), jnp.float32),
            pltpu.VMEM_SHARED((NACC, H), jnp.float32),
            pltpu.SemaphoreType.DMA,
            pltpu.SemaphoreType.DMA,
            pltpu.SemaphoreType.DMA,
            pltpu.SemaphoreType.DMA,
        ],
    )
    def k(hs_hbm, src_hbm, dst_hbm, z_hbm, out_hbm,
          sslab, dslab, rows0, rows1, acc, gs0, gs1, ss0, ss1):
        cid = lax.axis_index("c")
        sid = lax.axis_index("s")
        wid = sid * NC + cid
        r0 = sid * RPT

        pltpu.sync_copy(z_hbm, acc.at[pl.ds(r0, RPT)])
        pltpu.sync_copy(src_hbm.at[wid], sslab)
        pltpu.sync_copy(dst_hbm.at[wid], dslab)
        plsc.subcore_barrier()

        pltpu.async_copy(hs_hbm.at[sslab.at[0]], rows0, gs0)
        pltpu.async_copy(hs_hbm.at[sslab.at[1]], rows1, gs1)

        @pl.loop(0, NCHUNK, step=2)
        def _(j):
            pltpu.make_async_copy(hs_hbm.at[sslab.at[j]], rows0, gs0).wait()
            pltpu.async_copy(rows0, acc.at[dslab.at[j]], ss0, add=True)
            pltpu.make_async_copy(hs_hbm.at[sslab.at[j + 1]], rows1, gs1).wait()
            pltpu.async_copy(rows1, acc.at[dslab.at[j + 1]], ss1, add=True)
            pltpu.make_async_copy(rows0, acc.at[dslab.at[j]], ss0).wait()

            @pl.when(j + 2 < NCHUNK)
            def _():
                pltpu.async_copy(hs_hbm.at[sslab.at[j + 2]], rows0, gs0)

            pltpu.make_async_copy(rows1, acc.at[dslab.at[j + 1]], ss1).wait()

            @pl.when(j + 3 < NCHUNK)
            def _():
                pltpu.async_copy(hs_hbm.at[sslab.at[j + 3]], rows1, gs1)

        plsc.subcore_barrier()
        pltpu.sync_copy(acc.at[pl.ds(r0, RPT)], out_hbm.at[cid, pl.ds(r0, RPT)])

    return k(hs, srcr, dstr, zrows)


def _sc_deg(dstr, ones, zrows):
    @functools.partial(
        pl.kernel,
        out_type=jax.ShapeDtypeStruct((NC, NACC, DW), jnp.float32),
        mesh=_SC_MESH,
        compiler_params=pltpu.CompilerParams(use_tc_tiling_on_sc=False),
        scratch_types=[
            pltpu.VMEM((NCHUNK, CH), jnp.int32),
            pltpu.VMEM((CH, DW), jnp.float32),
            pltpu.VMEM_SHARED((NACC, DW), jnp.float32),
        ],
    )
    def k(dst_hbm, ones_hbm, z_hbm, out_hbm, dslab, ones_v, acc):
        cid = lax.axis_index("c")
        sid = lax.axis_index("s")
        wid = sid * NC + cid
        r0 = sid * RPT

        pltpu.sync_copy(z_hbm, acc.at[pl.ds(r0, RPT)])
        pltpu.sync_copy(ones_hbm, ones_v)
        pltpu.sync_copy(dst_hbm.at[wid], dslab)
        plsc.subcore_barrier()

        @pl.loop(0, NCHUNK)
        def _(j):
            pltpu.sync_copy(ones_v, acc.at[dslab.at[j]], add=True)

        plsc.subcore_barrier()
        pltpu.sync_copy(acc.at[pl.ds(r0, RPT)], out_hbm.at[cid, pl.ds(r0, RPT)])

    return k(dstr, ones, zrows)


def _tc_mm(x, w):
    def body(x_ref, w_ref, o_ref):
        o_ref[...] = jnp.dot(x_ref[...], w_ref[...],
                             preferred_element_type=jnp.float32)

    return pl.pallas_call(
        body, out_shape=jax.ShapeDtypeStruct((N, H), jnp.float32))(x, w)


def _tc_dinv_scale(z1, degp, batch2):
    def body(z_ref, d_ref, bt_ref, dinv_ref, hs_ref, co_ref):
        deg = 1.0 + d_ref[0, 0:N, 0:1] + d_ref[1, 0:N, 0:1]
        dinv = lax.rsqrt(deg)
        dinv_ref[...] = dinv
        hs_ref[...] = z_ref[...] * dinv
        bt = bt_ref[...]
        gi = lax.broadcasted_iota(jnp.int32, (N, G), 1)
        co_ref[0:1, :] = jnp.sum((bt == gi).astype(jnp.int32), axis=0,
                                 keepdims=True)
        co_ref[1:2, :] = jnp.sum((bt < gi).astype(jnp.int32), axis=0,
                                 keepdims=True)

    return pl.pallas_call(
        body,
        out_shape=(jax.ShapeDtypeStruct((N, 1), jnp.float32),
                   jax.ShapeDtypeStruct((N, H), jnp.float32),
                   jax.ShapeDtypeStruct((2, G), jnp.int32)))(z1, degp, batch2)


def _tc_layer(ap, hs, dinv, b, g, be, wn):
    def body(p_ref, hs_ref, dinv_ref, b_ref, g_ref, be_ref, w_ref, o_ref):
        dv = dinv_ref[...]
        agg = dv * (p_ref[0, 0:N, :] + p_ref[1, 0:N, :] + hs_ref[...]) + b_ref[...]
        m = jnp.mean(agg, axis=0, keepdims=True)
        c = agg - m
        v = jnp.mean(c * c, axis=0, keepdims=True)
        hn = jnp.maximum(c * lax.rsqrt(v + 1e-5) * g_ref[...] + be_ref[...],
                         0.0)
        z = jnp.dot(hn, w_ref[...], preferred_element_type=jnp.float32)
        o_ref[...] = z * dv

    return pl.pallas_call(
        body, out_shape=jax.ShapeDtypeStruct((N, H), jnp.float32))(
            ap, hs, dinv, b, g, be, wn)


def _tc_final(ap, hs, dinv, b, g, be, co, wo, bo):
    def body(p_ref, hs_ref, dinv_ref, b_ref, g_ref, be_ref, co_ref, wo_ref,
             bo_ref, o_ref, h3_s, mean_s, max_s):
        agg = (dinv_ref[...] * (p_ref[0, 0:N, :] + p_ref[1, 0:N, :]
                                + hs_ref[...]) + b_ref[...])
        m = jnp.mean(agg, axis=0, keepdims=True)
        c = agg - m
        v = jnp.mean(c * c, axis=0, keepdims=True)
        h3 = jnp.maximum(c * lax.rsqrt(v + 1e-5) * g_ref[...] + be_ref[...],
                         0.0)
        h3_s[0:N, :] = h3
        h3_s[N:NPAD, :] = jnp.zeros((NPAD - N, H), jnp.float32)

        def seg(gi, carry):
            cnt = co_ref[0, gi]
            off = co_ref[1, gi]
            nblk = (cnt + (BK - 1)) // BK

            def blk(i, sm):
                s, mx = sm
                rows = h3_s[pl.ds(off + i * BK, BK), :]
                rid = lax.broadcasted_iota(jnp.int32, (BK, 1), 0) + i * BK
                valid = rid < cnt
                s = s + jnp.sum(jnp.where(valid, rows, 0.0), axis=0,
                                keepdims=True)
                mx = jnp.maximum(mx, jnp.max(
                    jnp.where(valid, rows, -jnp.inf), axis=0, keepdims=True))
                return s, mx

            s0 = jnp.zeros((1, H), jnp.float32)
            m0 = jnp.full((1, H), -jnp.inf, jnp.float32)
            s, mx = lax.fori_loop(0, nblk, blk, (s0, m0))
            cntf = jnp.maximum(cnt, 1).astype(jnp.float32)
            mean_s[pl.ds(gi, 1), :] = s / cntf
            max_s[pl.ds(gi, 1), :] = jnp.where(cnt > 0, mx, 0.0)
            return carry

        lax.fori_loop(0, G, seg, 0)
        o_ref[...] = (
            jnp.dot(mean_s[...], wo_ref[0:H, :],
                    preferred_element_type=jnp.float32)
            + jnp.dot(max_s[...], wo_ref[H:2 * H, :],
                      preferred_element_type=jnp.float32)
            + bo_ref[...])

    vm = pl.BlockSpec(memory_space=pltpu.VMEM)
    return pl.pallas_call(
        body,
        out_shape=jax.ShapeDtypeStruct((G, 1), jnp.float32),
        in_specs=[vm, vm, vm, vm, vm, vm,
                  pl.BlockSpec(memory_space=pltpu.SMEM), vm, vm],
        scratch_shapes=[
            pltpu.VMEM((NPAD, H), jnp.float32),
            pltpu.VMEM((G, H), jnp.float32),
            pltpu.VMEM((G, H), jnp.float32),
        ])(ap, hs, dinv, b, g, be, co, wo, bo)


def kernel(x, edge_index, batch, W1, b1, W2, b2, W3, b3,
           g1, be1, g2, be2, g3, be3, Wo, bo):
    npad_e = E_PAD - E
    pad_i = jnp.arange(npad_e, dtype=jnp.int32)
    srcr = jnp.concatenate([edge_index[0], pad_i % N]).reshape(NW, NCHUNK, CH)
    dstr = jnp.concatenate([edge_index[1], N + pad_i % (NACC - N)]
                           ).reshape(NW, NCHUNK, CH)
    batch2 = batch.reshape(N, 1)
    b1r, b2r, b3r = (b.reshape(1, H) for b in (b1, b2, b3))
    g1r, g2r, g3r = (g.reshape(1, H) for g in (g1, g2, g3))
    be1r, be2r, be3r = (b.reshape(1, H) for b in (be1, be2, be3))
    bor = bo.reshape(1, 1)
    zH = jnp.zeros((RPT, H), jnp.float32)
    zD = jnp.zeros((RPT, DW), jnp.float32)
    onesD = jnp.ones((CH, DW), jnp.float32)

    degp = _sc_deg(dstr, onesD, zD)
    z1 = _tc_mm(x, W1)
    dinv, hs1, co = _tc_dinv_scale(z1, degp, batch2)

    a1 = _sc_agg(hs1, srcr, dstr, zH)
    hs2 = _tc_layer(a1, hs1, dinv, b1r, g1r, be1r, W2)
    a2 = _sc_agg(hs2, srcr, dstr, zH)
    hs3 = _tc_layer(a2, hs2, dinv, b2r, g2r, be2r, W3)
    a3 = _sc_agg(hs3, srcr, dstr, zH)

    return _tc_final(a3, hs3, dinv, b3r, g3r, be3r, co, Wo, bor)


# speedup vs baseline: 1.1481x; 1.1481x over previous
import functools

import jax
import jax.numpy as jnp
from jax import lax
from jax.experimental import pallas as pl
from jax.experimental.pallas import tpu as pltpu
from jax.experimental.pallas import tpu_sc as plsc

N = 10000
E = 320000
F_IN = 128
H = 64
G = 64

NC, NS = 2, 16
NW = NC * NS
CH = 128
NCHUNK = 80
EPW = NCHUNK * CH
E_PAD = NW * EPW
NACC = 10240
RPT = NACC // NS
DW = 16

BK = 240
NPAD = N + BK

_SC_MESH = plsc.VectorSubcoreMesh(core_axis_name="c", subcore_axis_name="s")


def _sc_agg(hs, srcr, dstr, zrows):
    @functools.partial(
        pl.kernel,
        out_type=jax.ShapeDtypeStruct((NC, NACC, H), jnp.float32),
        mesh=_SC_MESH,
        compiler_params=pltpu.CompilerParams(use_tc_tiling_on_sc=False),
        scratch_types=[
            pltpu.VMEM((NCHUNK, CH), jnp.int32),
            pltpu.VMEM((NCHUNK, CH), jnp.int32),
            pltpu.VMEM((CH, H), jnp.float32),
            pltpu.VMEM((CH, H), jnp.float32),
            pltpu.VMEM_SHARED((NACC, H), jnp.float32),
            pltpu.SemaphoreType.DMA,
            pltpu.SemaphoreType.DMA,
            pltpu.SemaphoreType.DMA,
            pltpu.SemaphoreType.DMA,
        ],
    )
    def k(hs_hbm, src_hbm, dst_hbm, z_hbm, out_hbm,
          sslab, dslab, rows0, rows1, acc, gs0, gs1, ss0, ss1):
        cid = lax.axis_index("c")
        sid = lax.axis_index("s")
        wid = sid * NC + cid
        r0 = sid * RPT

        pltpu.sync_copy(z_hbm, acc.at[pl.ds(r0, RPT)])
        pltpu.sync_copy(src_hbm.at[wid], sslab)
        pltpu.sync_copy(dst_hbm.at[wid], dslab)
        plsc.subcore_barrier()

        pltpu.async_copy(hs_hbm.at[sslab.at[0]], rows0, gs0)
        pltpu.async_copy(hs_hbm.at[sslab.at[1]], rows1, gs1)

        @pl.loop(0, NCHUNK, step=2)
        def _(j):
            pltpu.make_async_copy(hs_hbm.at[sslab.at[j]], rows0, gs0).wait()
            pltpu.sync_copy(rows0, acc.at[dslab.at[j]], add=True)

            @pl.when(j + 2 < NCHUNK)
            def _():
                pltpu.async_copy(hs_hbm.at[sslab.at[j + 2]], rows0, gs0)

            pltpu.make_async_copy(hs_hbm.at[sslab.at[j + 1]], rows1, gs1).wait()
            pltpu.sync_copy(rows1, acc.at[dslab.at[j + 1]], add=True)

            @pl.when(j + 3 < NCHUNK)
            def _():
                pltpu.async_copy(hs_hbm.at[sslab.at[j + 3]], rows1, gs1)

        plsc.subcore_barrier()
        pltpu.sync_copy(acc.at[pl.ds(r0, RPT)], out_hbm.at[cid, pl.ds(r0, RPT)])

    return k(hs, srcr, dstr, zrows)


def _sc_deg(dstr, ones, zrows):
    @functools.partial(
        pl.kernel,
        out_type=jax.ShapeDtypeStruct((NC, NACC, DW), jnp.float32),
        mesh=_SC_MESH,
        compiler_params=pltpu.CompilerParams(use_tc_tiling_on_sc=False),
        scratch_types=[
            pltpu.VMEM((NCHUNK, CH), jnp.int32),
            pltpu.VMEM((CH, DW), jnp.float32),
            pltpu.VMEM_SHARED((NACC, DW), jnp.float32),
        ],
    )
    def k(dst_hbm, ones_hbm, z_hbm, out_hbm, dslab, ones_v, acc):
        cid = lax.axis_index("c")
        sid = lax.axis_index("s")
        wid = sid * NC + cid
        r0 = sid * RPT

        pltpu.sync_copy(z_hbm, acc.at[pl.ds(r0, RPT)])
        pltpu.sync_copy(ones_hbm, ones_v)
        pltpu.sync_copy(dst_hbm.at[wid], dslab)
        plsc.subcore_barrier()

        @pl.loop(0, NCHUNK)
        def _(j):
            pltpu.sync_copy(ones_v, acc.at[dslab.at[j]], add=True)

        plsc.subcore_barrier()
        pltpu.sync_copy(acc.at[pl.ds(r0, RPT)], out_hbm.at[cid, pl.ds(r0, RPT)])

    return k(dstr, ones, zrows)


def _tc_mm(x, w):
    def body(x_ref, w_ref, o_ref):
        o_ref[...] = jnp.dot(x_ref[...], w_ref[...],
                             preferred_element_type=jnp.float32)

    return pl.pallas_call(
        body, out_shape=jax.ShapeDtypeStruct((N, H), jnp.float32))(x, w)


def _tc_dinv_scale(z1, degp, batch2):
    def body(z_ref, d_ref, bt_ref, dinv_ref, hs_ref, co_ref):
        deg = 1.0 + d_ref[0, 0:N, 0:1] + d_ref[1, 0:N, 0:1]
        dinv = lax.rsqrt(deg)
        dinv_ref[...] = dinv
        hs_ref[...] = z_ref[...] * dinv
        bt = bt_ref[...]
        gi = lax.broadcasted_iota(jnp.int32, (N, G), 1)
        co_ref[0:1, :] = jnp.sum((bt == gi).astype(jnp.int32), axis=0,
                                 keepdims=True)
        co_ref[1:2, :] = jnp.sum((bt < gi).astype(jnp.int32), axis=0,
                                 keepdims=True)

    return pl.pallas_call(
        body,
        out_shape=(jax.ShapeDtypeStruct((N, 1), jnp.float32),
                   jax.ShapeDtypeStruct((N, H), jnp.float32),
                   jax.ShapeDtypeStruct((2, G), jnp.int32)))(z1, degp, batch2)


def _tc_layer(ap, hs, dinv, b, g, be, wn):
    def body(p_ref, hs_ref, dinv_ref, b_ref, g_ref, be_ref, w_ref, o_ref):
        dv = dinv_ref[...]
        agg = dv * (p_ref[0, 0:N, :] + p_ref[1, 0:N, :] + hs_ref[...]) + b_ref[...]
        m = jnp.mean(agg, axis=0, keepdims=True)
        c = agg - m
        v = jnp.mean(c * c, axis=0, keepdims=True)
        hn = jnp.maximum(c * lax.rsqrt(v + 1e-5) * g_ref[...] + be_ref[...],
                         0.0)
        z = jnp.dot(hn, w_ref[...], preferred_element_type=jnp.float32)
        o_ref[...] = z * dv

    return pl.pallas_call(
        body, out_shape=jax.ShapeDtypeStruct((N, H), jnp.float32))(
            ap, hs, dinv, b, g, be, wn)


def _tc_final(ap, hs, dinv, b, g, be, co, wo, bo):
    def body(p_ref, hs_ref, dinv_ref, b_ref, g_ref, be_ref, co_ref, wo_ref,
             bo_ref, o_ref, h3_s, mean_s, max_s):
        agg = (dinv_ref[...] * (p_ref[0, 0:N, :] + p_ref[1, 0:N, :]
                                + hs_ref[...]) + b_ref[...])
        m = jnp.mean(agg, axis=0, keepdims=True)
        c = agg - m
        v = jnp.mean(c * c, axis=0, keepdims=True)
        h3 = jnp.maximum(c * lax.rsqrt(v + 1e-5) * g_ref[...] + be_ref[...],
                         0.0)
        h3_s[0:N, :] = h3
        h3_s[N:NPAD, :] = jnp.zeros((NPAD - N, H), jnp.float32)

        def seg(gi, carry):
            cnt = co_ref[0, gi]
            off = co_ref[1, gi]
            nblk = (cnt + (BK - 1)) // BK

            def blk(i, sm):
                s, mx = sm
                rows = h3_s[pl.ds(off + i * BK, BK), :]
                rid = lax.broadcasted_iota(jnp.int32, (BK, 1), 0) + i * BK
                valid = rid < cnt
                s = s + jnp.sum(jnp.where(valid, rows, 0.0), axis=0,
                                keepdims=True)
                mx = jnp.maximum(mx, jnp.max(
                    jnp.where(valid, rows, -jnp.inf), axis=0, keepdims=True))
                return s, mx

            s0 = jnp.zeros((1, H), jnp.float32)
            m0 = jnp.full((1, H), -jnp.inf, jnp.float32)
            s, mx = lax.fori_loop(0, nblk, blk, (s0, m0))
            cntf = jnp.maximum(cnt, 1).astype(jnp.float32)
            mean_s[pl.ds(gi, 1), :] = s / cntf
            max_s[pl.ds(gi, 1), :] = jnp.where(cnt > 0, mx, 0.0)
            return carry

        lax.fori_loop(0, G, seg, 0)
        o_ref[...] = (
            jnp.dot(mean_s[...], wo_ref[0:H, :],
                    preferred_element_type=jnp.float32)
            + jnp.dot(max_s[...], wo_ref[H:2 * H, :],
                      preferred_element_type=jnp.float32)
            + bo_ref[...])

    vm = pl.BlockSpec(memory_space=pltpu.VMEM)
    return pl.pallas_call(
        body,
        out_shape=jax.ShapeDtypeStruct((G, 1), jnp.float32),
        in_specs=[vm, vm, vm, vm, vm, vm,
                  pl.BlockSpec(memory_space=pltpu.SMEM), vm, vm],
        scratch_shapes=[
            pltpu.VMEM((NPAD, H), jnp.float32),
            pltpu.VMEM((G, H), jnp.float32),
            pltpu.VMEM((G, H), jnp.float32),
        ])(ap, hs, dinv, b, g, be, co, wo, bo)


def kernel(x, edge_index, batch, W1, b1, W2, b2, W3, b3,
           g1, be1, g2, be2, g3, be3, Wo, bo):
    npad_e = E_PAD - E
    pad_i = jnp.arange(npad_e, dtype=jnp.int32)
    srcr = jnp.concatenate([edge_index[0], pad_i % N]).reshape(NW, NCHUNK, CH)
    dstr = jnp.concatenate([edge_index[1], N + pad_i % (NACC - N)]
                           ).reshape(NW, NCHUNK, CH)
    batch2 = batch.reshape(N, 1)
    b1r, b2r, b3r = (b.reshape(1, H) for b in (b1, b2, b3))
    g1r, g2r, g3r = (g.reshape(1, H) for g in (g1, g2, g3))
    be1r, be2r, be3r = (b.reshape(1, H) for b in (be1, be2, be3))
    bor = bo.reshape(1, 1)
    zH = jnp.zeros((RPT, H), jnp.float32)
    zD = jnp.zeros((RPT, DW), jnp.float32)
    onesD = jnp.ones((CH, DW), jnp.float32)

    degp = _sc_deg(dstr, onesD, zD)
    z1 = _tc_mm(x, W1)
    dinv, hs1, co = _tc_dinv_scale(z1, degp, batch2)

    a1 = _sc_agg(hs1, srcr, dstr, zH)
    hs2 = _tc_layer(a1, hs1, dinv, b1r, g1r, be1r, W2)
    a2 = _sc_agg(hs2, srcr, dstr, zH)
    hs3 = _tc_layer(a2, hs2, dinv, b2r, g2r, be2r, W3)
    a3 = _sc_agg(hs3, srcr, dstr, zH)

    return _tc_final(a3, hs3, dinv, b3r, g3r, be3r, co, Wo, bor)


# speedup vs baseline: 1.1536x; 1.0048x over previous
import functools

import jax
import jax.numpy as jnp
from jax import lax
from jax.experimental import pallas as pl
from jax.experimental.pallas import tpu as pltpu
from jax.experimental.pallas import tpu_sc as plsc

N = 10000
E = 320000
F_IN = 128
H = 64
G = 64

NC, NS = 2, 16
NW = NC * NS
CH = 128
NCHUNK = 80
EPW = NCHUNK * CH
E_PAD = NW * EPW
NACC = 10240
RPT = NACC // NS
DW = 16

BK = 240
NPAD = N + BK

_SC_MESH = plsc.VectorSubcoreMesh(core_axis_name="c", subcore_axis_name="s")


def _sc_agg(hs, srcr, dstr, zrows):
    @functools.partial(
        pl.kernel,
        out_type=jax.ShapeDtypeStruct((NC, NACC, H), jnp.float32),
        mesh=_SC_MESH,
        compiler_params=pltpu.CompilerParams(use_tc_tiling_on_sc=False),
        scratch_types=[
            pltpu.VMEM((NCHUNK, CH), jnp.int32),
            pltpu.VMEM((NCHUNK, CH), jnp.int32),
            pltpu.VMEM((CH, H), jnp.float32),
            pltpu.VMEM((CH, H), jnp.float32),
            pltpu.VMEM_SHARED((NACC, H), jnp.float32),
            pltpu.SemaphoreType.DMA,
            pltpu.SemaphoreType.DMA,
            pltpu.SemaphoreType.DMA,
            pltpu.SemaphoreType.DMA,
        ],
    )
    def k(hs_hbm, src_hbm, dst_hbm, z_hbm, out_hbm,
          sslab, dslab, rows0, rows1, acc, gs0, gs1, ss0, ss1):
        cid = lax.axis_index("c")
        sid = lax.axis_index("s")
        wid = sid * NC + cid
        r0 = sid * RPT

        pltpu.sync_copy(z_hbm, acc.at[pl.ds(r0, RPT)])
        pltpu.sync_copy(src_hbm.at[wid], sslab)
        pltpu.sync_copy(dst_hbm.at[wid], dslab)
        plsc.subcore_barrier()

        pltpu.async_copy(hs_hbm.at[sslab.at[0]], rows0, gs0)
        pltpu.async_copy(hs_hbm.at[sslab.at[1]], rows1, gs1)

        @pl.loop(0, NCHUNK, step=2)
        def _(j):
            pltpu.make_async_copy(hs_hbm.at[sslab.at[j]], rows0, gs0).wait()
            pltpu.sync_copy(rows0, acc.at[dslab.at[j]], add=True)

            @pl.when(j + 2 < NCHUNK)
            def _():
                pltpu.async_copy(hs_hbm.at[sslab.at[j + 2]], rows0, gs0)

            pltpu.make_async_copy(hs_hbm.at[sslab.at[j + 1]], rows1, gs1).wait()
            pltpu.sync_copy(rows1, acc.at[dslab.at[j + 1]], add=True)

            @pl.when(j + 3 < NCHUNK)
            def _():
                pltpu.async_copy(hs_hbm.at[sslab.at[j + 3]], rows1, gs1)

        plsc.subcore_barrier()
        pltpu.sync_copy(acc.at[pl.ds(r0, RPT)], out_hbm.at[cid, pl.ds(r0, RPT)])

    return k(hs, srcr, dstr, zrows)


def _sc_deg(dstr, ones, zrows):
    @functools.partial(
        pl.kernel,
        out_type=jax.ShapeDtypeStruct((NC, NACC, DW), jnp.float32),
        mesh=_SC_MESH,
        compiler_params=pltpu.CompilerParams(use_tc_tiling_on_sc=False),
        scratch_types=[
            pltpu.VMEM((NCHUNK, CH), jnp.int32),
            pltpu.VMEM((CH, DW), jnp.float32),
            pltpu.VMEM_SHARED((NACC, DW), jnp.float32),
            pltpu.SemaphoreType.DMA,
        ],
    )
    def k(dst_hbm, ones_hbm, z_hbm, out_hbm, dslab, ones_v, acc, sem):
        cid = lax.axis_index("c")
        sid = lax.axis_index("s")
        wid = sid * NC + cid
        r0 = sid * RPT

        pltpu.sync_copy(z_hbm, acc.at[pl.ds(r0, RPT)])
        pltpu.sync_copy(ones_hbm, ones_v)
        pltpu.sync_copy(dst_hbm.at[wid], dslab)
        plsc.subcore_barrier()

        @pl.loop(0, NCHUNK, step=4)
        def _(j):
            pltpu.async_copy(ones_v, acc.at[dslab.at[j]], sem, add=True)
            pltpu.async_copy(ones_v, acc.at[dslab.at[j + 1]], sem, add=True)
            pltpu.async_copy(ones_v, acc.at[dslab.at[j + 2]], sem, add=True)
            pltpu.async_copy(ones_v, acc.at[dslab.at[j + 3]], sem, add=True)
            pltpu.make_async_copy(ones_v, acc.at[dslab.at[j]], sem).wait()
            pltpu.make_async_copy(ones_v, acc.at[dslab.at[j + 1]], sem).wait()
            pltpu.make_async_copy(ones_v, acc.at[dslab.at[j + 2]], sem).wait()
            pltpu.make_async_copy(ones_v, acc.at[dslab.at[j + 3]], sem).wait()

        plsc.subcore_barrier()
        pltpu.sync_copy(acc.at[pl.ds(r0, RPT)], out_hbm.at[cid, pl.ds(r0, RPT)])

    return k(dstr, ones, zrows)


def _tc_mm(x, w):
    def body(x_ref, w_ref, o_ref):
        o_ref[...] = jnp.dot(x_ref[...], w_ref[...],
                             preferred_element_type=jnp.float32)

    return pl.pallas_call(
        body, out_shape=jax.ShapeDtypeStruct((N, H), jnp.float32))(x, w)


def _tc_dinv_scale(z1, degp, batch2):
    def body(z_ref, d_ref, bt_ref, dinv_ref, hs_ref, co_ref):
        deg = 1.0 + d_ref[0, 0:N, 0:1] + d_ref[1, 0:N, 0:1]
        dinv = lax.rsqrt(deg)
        dinv_ref[...] = dinv
        hs_ref[...] = z_ref[...] * dinv
        bt = bt_ref[...]
        gi = lax.broadcasted_iota(jnp.int32, (N, G), 1)
        co_ref[0:1, :] = jnp.sum((bt == gi).astype(jnp.int32), axis=0,
                                 keepdims=True)
        co_ref[1:2, :] = jnp.sum((bt < gi).astype(jnp.int32), axis=0,
                                 keepdims=True)

    return pl.pallas_call(
        body,
        out_shape=(jax.ShapeDtypeStruct((N, 1), jnp.float32),
                   jax.ShapeDtypeStruct((N, H), jnp.float32),
                   jax.ShapeDtypeStruct((2, G), jnp.int32)))(z1, degp, batch2)


def _tc_layer(ap, hs, dinv, b, g, be, wn):
    def body(p_ref, hs_ref, dinv_ref, b_ref, g_ref, be_ref, w_ref, o_ref):
        dv = dinv_ref[...]
        agg = dv * (p_ref[0, 0:N, :] + p_ref[1, 0:N, :] + hs_ref[...]) + b_ref[...]
        m = jnp.mean(agg, axis=0, keepdims=True)
        c = agg - m
        v = jnp.mean(c * c, axis=0, keepdims=True)
        hn = jnp.maximum(c * lax.rsqrt(v + 1e-5) * g_ref[...] + be_ref[...],
                         0.0)
        z = jnp.dot(hn, w_ref[...], preferred_element_type=jnp.float32)
        o_ref[...] = z * dv

    return pl.pallas_call(
        body, out_shape=jax.ShapeDtypeStruct((N, H), jnp.float32))(
            ap, hs, dinv, b, g, be, wn)


def _tc_final(ap, hs, dinv, b, g, be, co, wo, bo):
    def body(p_ref, hs_ref, dinv_ref, b_ref, g_ref, be_ref, co_ref, wo_ref,
             bo_ref, o_ref, h3_s, mean_s, max_s):
        agg = (dinv_ref[...] * (p_ref[0, 0:N, :] + p_ref[1, 0:N, :]
                                + hs_ref[...]) + b_ref[...])
        m = jnp.mean(agg, axis=0, keepdims=True)
        c = agg - m
        v = jnp.mean(c * c, axis=0, keepdims=True)
        h3 = jnp.maximum(c * lax.rsqrt(v + 1e-5) * g_ref[...] + be_ref[...],
                         0.0)
        h3_s[0:N, :] = h3
        h3_s[N:NPAD, :] = jnp.zeros((NPAD - N, H), jnp.float32)

        def seg(gi, carry):
            cnt = co_ref[0, gi]
            off = co_ref[1, gi]
            nblk = (cnt + (BK - 1)) // BK

            def blk(i, sm):
                s, mx = sm
                rows = h3_s[pl.ds(off + i * BK, BK), :]
                rid = lax.broadcasted_iota(jnp.int32, (BK, 1), 0) + i * BK
                valid = rid < cnt
                s = s + jnp.sum(jnp.where(valid, rows, 0.0), axis=0,
                                keepdims=True)
                mx = jnp.maximum(mx, jnp.max(
                    jnp.where(valid, rows, -jnp.inf), axis=0, keepdims=True))
                return s, mx

            s0 = jnp.zeros((1, H), jnp.float32)
            m0 = jnp.full((1, H), -jnp.inf, jnp.float32)
            s, mx = lax.fori_loop(0, nblk, blk, (s0, m0))
            cntf = jnp.maximum(cnt, 1).astype(jnp.float32)
            mean_s[pl.ds(gi, 1), :] = s / cntf
            max_s[pl.ds(gi, 1), :] = jnp.where(cnt > 0, mx, 0.0)
            return carry

        lax.fori_loop(0, G, seg, 0)
        o_ref[...] = (
            jnp.dot(mean_s[...], wo_ref[0:H, :],
                    preferred_element_type=jnp.float32)
            + jnp.dot(max_s[...], wo_ref[H:2 * H, :],
                      preferred_element_type=jnp.float32)
            + bo_ref[...])

    vm = pl.BlockSpec(memory_space=pltpu.VMEM)
    return pl.pallas_call(
        body,
        out_shape=jax.ShapeDtypeStruct((G, 1), jnp.float32),
        in_specs=[vm, vm, vm, vm, vm, vm,
                  pl.BlockSpec(memory_space=pltpu.SMEM), vm, vm],
        scratch_shapes=[
            pltpu.VMEM((NPAD, H), jnp.float32),
            pltpu.VMEM((G, H), jnp.float32),
            pltpu.VMEM((G, H), jnp.float32),
        ])(ap, hs, dinv, b, g, be, co, wo, bo)


def kernel(x, edge_index, batch, W1, b1, W2, b2, W3, b3,
           g1, be1, g2, be2, g3, be3, Wo, bo):
    import numpy as _np
    npad_e = E_PAD - E
    pad_i = _np.arange(npad_e, dtype=_np.int32)
    pad_src = jnp.asarray(pad_i % N)
    pad_dst = jnp.asarray(N + pad_i % (NACC - N))
    srcr = jnp.concatenate([edge_index[0], pad_src]).reshape(NW, NCHUNK, CH)
    dstr = jnp.concatenate([edge_index[1], pad_dst]).reshape(NW, NCHUNK, CH)
    batch2 = batch.reshape(N, 1)
    b1r, b2r, b3r = (b.reshape(1, H) for b in (b1, b2, b3))
    g1r, g2r, g3r = (g.reshape(1, H) for g in (g1, g2, g3))
    be1r, be2r, be3r = (b.reshape(1, H) for b in (be1, be2, be3))
    bor = bo.reshape(1, 1)
    zH = jnp.zeros((RPT, H), jnp.float32)
    zD = jnp.zeros((RPT, DW), jnp.float32)
    onesD = jnp.ones((CH, DW), jnp.float32)

    degp = _sc_deg(dstr, onesD, zD)
    z1 = _tc_mm(x, W1)
    dinv, hs1, co = _tc_dinv_scale(z1, degp, batch2)

    a1 = _sc_agg(hs1, srcr, dstr, zH)
    hs2 = _tc_layer(a1, hs1, dinv, b1r, g1r, be1r, W2)
    a2 = _sc_agg(hs2, srcr, dstr, zH)
    hs3 = _tc_layer(a2, hs2, dinv, b2r, g2r, be2r, W3)
    a3 = _sc_agg(hs3, srcr, dstr, zH)

    return _tc_final(a3, hs3, dinv, b3r, g3r, be3r, co, Wo, bor)


# speedup vs baseline: 1.2228x; 1.0600x over previous
import functools

import jax
import jax.numpy as jnp
from jax import lax
from jax.experimental import pallas as pl
from jax.experimental.pallas import tpu as pltpu
from jax.experimental.pallas import tpu_sc as plsc

N = 10000
E = 320000
F_IN = 128
H = 64
G = 64

NC, NS = 2, 16
NW = NC * NS
CH = 128
NCHUNK = 80
EPW = NCHUNK * CH
E_PAD = NW * EPW
NACC = 10240
RPT = NACC // NS
DW = 16

BK = 240
NPAD = N + BK

_SC_MESH = plsc.VectorSubcoreMesh(core_axis_name="c", subcore_axis_name="s")


def _sc_agg(hs, srcr, dstr, zrows):
    @functools.partial(
        pl.kernel,
        out_type=jax.ShapeDtypeStruct((NC, NACC, H), jnp.float32),
        mesh=_SC_MESH,
        compiler_params=pltpu.CompilerParams(use_tc_tiling_on_sc=False),
        scratch_types=[
            pltpu.VMEM((NCHUNK, CH), jnp.int32),
            pltpu.VMEM((NCHUNK, CH), jnp.int32),
            pltpu.VMEM((CH, H), jnp.float32),
            pltpu.VMEM((CH, H), jnp.float32),
            pltpu.VMEM((CH, H), jnp.float32),
            pltpu.VMEM((CH, H), jnp.float32),
            pltpu.VMEM_SHARED((NACC, H), jnp.float32),
            pltpu.SemaphoreType.DMA,
            pltpu.SemaphoreType.DMA,
            pltpu.SemaphoreType.DMA,
            pltpu.SemaphoreType.DMA,
            pltpu.SemaphoreType.DMA,
        ],
    )
    def k(hs_hbm, src_hbm, dst_hbm, z_hbm, out_hbm,
          sslab, dslab, rows0, rows1, rows2, rows3, acc,
          gs0, gs1, gs2, gs3, ssem):
        cid = lax.axis_index("c")
        sid = lax.axis_index("s")
        wid = sid * NC + cid
        r0 = sid * RPT

        pltpu.sync_copy(z_hbm, acc.at[pl.ds(r0, RPT)])
        pltpu.sync_copy(src_hbm.at[wid], sslab)
        pltpu.sync_copy(dst_hbm.at[wid], dslab)
        plsc.subcore_barrier()

        rbuf = (rows0, rows1, rows2, rows3)
        gsem = (gs0, gs1, gs2, gs3)
        for b in range(4):
            pltpu.async_copy(hs_hbm.at[sslab.at[b]], rbuf[b], gsem[b])

        @pl.loop(0, NCHUNK, step=4)
        def _(j):
            pltpu.make_async_copy(hs_hbm.at[sslab.at[j]], rows0, gs0).wait()
            pltpu.make_async_copy(hs_hbm.at[sslab.at[j + 1]], rows1, gs1).wait()
            pltpu.async_copy(rows0, acc.at[dslab.at[j]], ssem, add=True)
            pltpu.async_copy(rows1, acc.at[dslab.at[j + 1]], ssem, add=True)
            pltpu.make_async_copy(hs_hbm.at[sslab.at[j + 2]], rows2, gs2).wait()
            pltpu.make_async_copy(hs_hbm.at[sslab.at[j + 3]], rows3, gs3).wait()
            pltpu.make_async_copy(rows0, acc.at[dslab.at[j]], ssem).wait()
            pltpu.make_async_copy(rows1, acc.at[dslab.at[j + 1]], ssem).wait()
            pltpu.async_copy(rows2, acc.at[dslab.at[j + 2]], ssem, add=True)
            pltpu.async_copy(rows3, acc.at[dslab.at[j + 3]], ssem, add=True)

            @pl.when(j + 4 < NCHUNK)
            def _():
                pltpu.async_copy(hs_hbm.at[sslab.at[j + 4]], rows0, gs0)

            @pl.when(j + 5 < NCHUNK)
            def _():
                pltpu.async_copy(hs_hbm.at[sslab.at[j + 5]], rows1, gs1)

            pltpu.make_async_copy(rows2, acc.at[dslab.at[j + 2]], ssem).wait()
            pltpu.make_async_copy(rows3, acc.at[dslab.at[j + 3]], ssem).wait()

            @pl.when(j + 6 < NCHUNK)
            def _():
                pltpu.async_copy(hs_hbm.at[sslab.at[j + 6]], rows2, gs2)

            @pl.when(j + 7 < NCHUNK)
            def _():
                pltpu.async_copy(hs_hbm.at[sslab.at[j + 7]], rows3, gs3)

        plsc.subcore_barrier()
        pltpu.sync_copy(acc.at[pl.ds(r0, RPT)], out_hbm.at[cid, pl.ds(r0, RPT)])

    return k(hs, srcr, dstr, zrows)


def _sc_deg(dstr, ones, zrows):
    @functools.partial(
        pl.kernel,
        out_type=jax.ShapeDtypeStruct((NC, NACC, DW), jnp.float32),
        mesh=_SC_MESH,
        compiler_params=pltpu.CompilerParams(use_tc_tiling_on_sc=False),
        scratch_types=[
            pltpu.VMEM((NCHUNK, CH), jnp.int32),
            pltpu.VMEM((CH, DW), jnp.float32),
            pltpu.VMEM_SHARED((NACC, DW), jnp.float32),
            pltpu.SemaphoreType.DMA,
        ],
    )
    def k(dst_hbm, ones_hbm, z_hbm, out_hbm, dslab, ones_v, acc, sem):
        cid = lax.axis_index("c")
        sid = lax.axis_index("s")
        wid = sid * NC + cid
        r0 = sid * RPT

        pltpu.sync_copy(z_hbm, acc.at[pl.ds(r0, RPT)])
        pltpu.sync_copy(ones_hbm, ones_v)
        pltpu.sync_copy(dst_hbm.at[wid], dslab)
        plsc.subcore_barrier()

        @pl.loop(0, NCHUNK, step=4)
        def _(j):
            pltpu.async_copy(ones_v, acc.at[dslab.at[j]], sem, add=True)
            pltpu.async_copy(ones_v, acc.at[dslab.at[j + 1]], sem, add=True)
            pltpu.async_copy(ones_v, acc.at[dslab.at[j + 2]], sem, add=True)
            pltpu.async_copy(ones_v, acc.at[dslab.at[j + 3]], sem, add=True)
            pltpu.make_async_copy(ones_v, acc.at[dslab.at[j]], sem).wait()
            pltpu.make_async_copy(ones_v, acc.at[dslab.at[j + 1]], sem).wait()
            pltpu.make_async_copy(ones_v, acc.at[dslab.at[j + 2]], sem).wait()
            pltpu.make_async_copy(ones_v, acc.at[dslab.at[j + 3]], sem).wait()

        plsc.subcore_barrier()
        pltpu.sync_copy(acc.at[pl.ds(r0, RPT)], out_hbm.at[cid, pl.ds(r0, RPT)])

    return k(dstr, ones, zrows)


def _tc_mm(x, w):
    def body(x_ref, w_ref, o_ref):
        o_ref[...] = jnp.dot(x_ref[...], w_ref[...],
                             preferred_element_type=jnp.float32)

    return pl.pallas_call(
        body, out_shape=jax.ShapeDtypeStruct((N, H), jnp.float32))(x, w)


def _tc_dinv_scale(z1, degp, batch2):
    def body(z_ref, d_ref, bt_ref, dinv_ref, hs_ref, co_ref):
        deg = 1.0 + d_ref[0, 0:N, 0:1] + d_ref[1, 0:N, 0:1]
        dinv = lax.rsqrt(deg)
        dinv_ref[...] = dinv
        hs_ref[...] = z_ref[...] * dinv
        bt = bt_ref[...]
        gi = lax.broadcasted_iota(jnp.int32, (N, G), 1)
        co_ref[0:1, :] = jnp.sum((bt == gi).astype(jnp.int32), axis=0,
                                 keepdims=True)
        co_ref[1:2, :] = jnp.sum((bt < gi).astype(jnp.int32), axis=0,
                                 keepdims=True)

    return pl.pallas_call(
        body,
        out_shape=(jax.ShapeDtypeStruct((N, 1), jnp.float32),
                   jax.ShapeDtypeStruct((N, H), jnp.float32),
                   jax.ShapeDtypeStruct((2, G), jnp.int32)))(z1, degp, batch2)


def _tc_layer(ap, hs, dinv, b, g, be, wn):
    def body(p_ref, hs_ref, dinv_ref, b_ref, g_ref, be_ref, w_ref, o_ref):
        dv = dinv_ref[...]
        agg = dv * (p_ref[0, 0:N, :] + p_ref[1, 0:N, :] + hs_ref[...]) + b_ref[...]
        m = jnp.mean(agg, axis=0, keepdims=True)
        c = agg - m
        v = jnp.mean(c * c, axis=0, keepdims=True)
        hn = jnp.maximum(c * lax.rsqrt(v + 1e-5) * g_ref[...] + be_ref[...],
                         0.0)
        z = jnp.dot(hn, w_ref[...], preferred_element_type=jnp.float32)
        o_ref[...] = z * dv

    return pl.pallas_call(
        body, out_shape=jax.ShapeDtypeStruct((N, H), jnp.float32))(
            ap, hs, dinv, b, g, be, wn)


def _tc_final(ap, hs, dinv, b, g, be, co, wo, bo):
    def body(p_ref, hs_ref, dinv_ref, b_ref, g_ref, be_ref, co_ref, wo_ref,
             bo_ref, o_ref, h3_s, mean_s, max_s):
        agg = (dinv_ref[...] * (p_ref[0, 0:N, :] + p_ref[1, 0:N, :]
                                + hs_ref[...]) + b_ref[...])
        m = jnp.mean(agg, axis=0, keepdims=True)
        c = agg - m
        v = jnp.mean(c * c, axis=0, keepdims=True)
        h3 = jnp.maximum(c * lax.rsqrt(v + 1e-5) * g_ref[...] + be_ref[...],
                         0.0)
        h3_s[0:N, :] = h3
        h3_s[N:NPAD, :] = jnp.zeros((NPAD - N, H), jnp.float32)

        def seg(gi, carry):
            cnt = co_ref[0, gi]
            off = co_ref[1, gi]
            nblk = (cnt + (BK - 1)) // BK

            def blk(i, sm):
                s, mx = sm
                rows = h3_s[pl.ds(off + i * BK, BK), :]
                rid = lax.broadcasted_iota(jnp.int32, (BK, 1), 0) + i * BK
                valid = rid < cnt
                s = s + jnp.sum(jnp.where(valid, rows, 0.0), axis=0,
                                keepdims=True)
                mx = jnp.maximum(mx, jnp.max(
                    jnp.where(valid, rows, -jnp.inf), axis=0, keepdims=True))
                return s, mx

            s0 = jnp.zeros((1, H), jnp.float32)
            m0 = jnp.full((1, H), -jnp.inf, jnp.float32)
            s, mx = lax.fori_loop(0, nblk, blk, (s0, m0))
            cntf = jnp.maximum(cnt, 1).astype(jnp.float32)
            mean_s[pl.ds(gi, 1), :] = s / cntf
            max_s[pl.ds(gi, 1), :] = jnp.where(cnt > 0, mx, 0.0)
            return carry

        lax.fori_loop(0, G, seg, 0)
        o_ref[...] = (
            jnp.dot(mean_s[...], wo_ref[0:H, :],
                    preferred_element_type=jnp.float32)
            + jnp.dot(max_s[...], wo_ref[H:2 * H, :],
                      preferred_element_type=jnp.float32)
            + bo_ref[...])

    vm = pl.BlockSpec(memory_space=pltpu.VMEM)
    return pl.pallas_call(
        body,
        out_shape=jax.ShapeDtypeStruct((G, 1), jnp.float32),
        in_specs=[vm, vm, vm, vm, vm, vm,
                  pl.BlockSpec(memory_space=pltpu.SMEM), vm, vm],
        scratch_shapes=[
            pltpu.VMEM((NPAD, H), jnp.float32),
            pltpu.VMEM((G, H), jnp.float32),
            pltpu.VMEM((G, H), jnp.float32),
        ])(ap, hs, dinv, b, g, be, co, wo, bo)


def kernel(x, edge_index, batch, W1, b1, W2, b2, W3, b3,
           g1, be1, g2, be2, g3, be3, Wo, bo):
    import numpy as _np
    npad_e = E_PAD - E
    pad_i = _np.arange(npad_e, dtype=_np.int32)
    pad_src = jnp.asarray(pad_i % N)
    pad_dst = jnp.asarray(N + pad_i % (NACC - N))
    srcr = jnp.concatenate([edge_index[0], pad_src]).reshape(NW, NCHUNK, CH)
    dstr = jnp.concatenate([edge_index[1], pad_dst]).reshape(NW, NCHUNK, CH)
    batch2 = batch.reshape(N, 1)
    b1r, b2r, b3r = (b.reshape(1, H) for b in (b1, b2, b3))
    g1r, g2r, g3r = (g.reshape(1, H) for g in (g1, g2, g3))
    be1r, be2r, be3r = (b.reshape(1, H) for b in (be1, be2, be3))
    bor = bo.reshape(1, 1)
    zH = jnp.zeros((RPT, H), jnp.float32)
    zD = jnp.zeros((RPT, DW), jnp.float32)
    onesD = jnp.ones((CH, DW), jnp.float32)

    degp = _sc_deg(dstr, onesD, zD)
    z1 = _tc_mm(x, W1)
    dinv, hs1, co = _tc_dinv_scale(z1, degp, batch2)

    a1 = _sc_agg(hs1, srcr, dstr, zH)
    hs2 = _tc_layer(a1, hs1, dinv, b1r, g1r, be1r, W2)
    a2 = _sc_agg(hs2, srcr, dstr, zH)
    hs3 = _tc_layer(a2, hs2, dinv, b2r, g2r, be2r, W3)
    a3 = _sc_agg(hs3, srcr, dstr, zH)

    return _tc_final(a3, hs3, dinv, b3r, g3r, be3r, co, Wo, bor)


# speedup vs baseline: 1.4894x; 1.2180x over previous
import functools

import jax
import jax.numpy as jnp
from jax import lax
from jax.experimental import pallas as pl
from jax.experimental.pallas import tpu as pltpu
from jax.experimental.pallas import tpu_sc as plsc

N = 10000
E = 320000
F_IN = 128
H = 64
G = 64

NC, NS = 2, 16
NW = NC * NS
CH = 128
NCHUNK = 80
EPW = NCHUNK * CH
E_PAD = NW * EPW
NACC = 10240
RPT = NACC // NS
DW = 16

BK = 240
NPAD = N + BK

_SC_MESH = plsc.VectorSubcoreMesh(core_axis_name="c", subcore_axis_name="s")


def _sc_agg(hs, srcr, dstr, zrows):
    @functools.partial(
        pl.kernel,
        out_type=jax.ShapeDtypeStruct((NC, NACC, H), jnp.float32),
        mesh=_SC_MESH,
        compiler_params=pltpu.CompilerParams(use_tc_tiling_on_sc=False),
        scratch_types=[
            pltpu.VMEM((NCHUNK, CH), jnp.int32),
            pltpu.VMEM((NCHUNK, CH), jnp.int32),
            pltpu.VMEM((CH, H), jnp.float32),
            pltpu.VMEM((CH, H), jnp.float32),
            pltpu.VMEM((CH, H), jnp.float32),
            pltpu.VMEM((CH, H), jnp.float32),
            pltpu.VMEM_SHARED((NACC, H), jnp.float32),
            pltpu.SemaphoreType.DMA,
            pltpu.SemaphoreType.DMA,
            pltpu.SemaphoreType.DMA,
            pltpu.SemaphoreType.DMA,
            pltpu.SemaphoreType.DMA,
        ],
    )
    def k(hs_hbm, src_hbm, dst_hbm, z_hbm, out_hbm,
          sslab, dslab, rows0, rows1, rows2, rows3, acc,
          gs0, gs1, gs2, gs3, ssem):
        cid = lax.axis_index("c")
        sid = lax.axis_index("s")
        wid = sid * NC + cid
        r0 = sid * RPT

        pltpu.sync_copy(z_hbm, acc.at[pl.ds(r0, RPT)])
        pltpu.sync_copy(src_hbm.at[wid], sslab)
        pltpu.sync_copy(dst_hbm.at[wid], dslab)
        plsc.subcore_barrier()

        rbuf = (rows0, rows1, rows2, rows3)
        gsem = (gs0, gs1, gs2, gs3)
        for b in range(4):
            pltpu.async_copy(hs_hbm.at[sslab.at[b]], rbuf[b], gsem[b])

        @pl.loop(0, NCHUNK, step=4)
        def _(j):
            pltpu.make_async_copy(hs_hbm.at[sslab.at[j]], rows0, gs0).wait()
            pltpu.make_async_copy(hs_hbm.at[sslab.at[j + 1]], rows1, gs1).wait()
            pltpu.async_copy(rows0, acc.at[dslab.at[j]], ssem, add=True)
            pltpu.async_copy(rows1, acc.at[dslab.at[j + 1]], ssem, add=True)
            pltpu.make_async_copy(hs_hbm.at[sslab.at[j + 2]], rows2, gs2).wait()
            pltpu.make_async_copy(hs_hbm.at[sslab.at[j + 3]], rows3, gs3).wait()
            pltpu.make_async_copy(rows0, acc.at[dslab.at[j]], ssem).wait()
            pltpu.make_async_copy(rows1, acc.at[dslab.at[j + 1]], ssem).wait()
            pltpu.async_copy(rows2, acc.at[dslab.at[j + 2]], ssem, add=True)
            pltpu.async_copy(rows3, acc.at[dslab.at[j + 3]], ssem, add=True)

            @pl.when(j + 4 < NCHUNK)
            def _():
                pltpu.async_copy(hs_hbm.at[sslab.at[j + 4]], rows0, gs0)

            @pl.when(j + 5 < NCHUNK)
            def _():
                pltpu.async_copy(hs_hbm.at[sslab.at[j + 5]], rows1, gs1)

            pltpu.make_async_copy(rows2, acc.at[dslab.at[j + 2]], ssem).wait()
            pltpu.make_async_copy(rows3, acc.at[dslab.at[j + 3]], ssem).wait()

            @pl.when(j + 6 < NCHUNK)
            def _():
                pltpu.async_copy(hs_hbm.at[sslab.at[j + 6]], rows2, gs2)

            @pl.when(j + 7 < NCHUNK)
            def _():
                pltpu.async_copy(hs_hbm.at[sslab.at[j + 7]], rows3, gs3)

        plsc.subcore_barrier()
        pltpu.sync_copy(acc.at[pl.ds(r0, RPT)], out_hbm.at[cid, pl.ds(r0, RPT)])

    return k(hs, srcr, dstr, zrows)


def _sc_deg(dstr, ones, zrows):
    @functools.partial(
        pl.kernel,
        out_type=jax.ShapeDtypeStruct((NC, NACC, DW), jnp.float32),
        mesh=_SC_MESH,
        compiler_params=pltpu.CompilerParams(use_tc_tiling_on_sc=False),
        scratch_types=[
            pltpu.VMEM((NCHUNK, CH), jnp.int32),
            pltpu.VMEM((CH, DW), jnp.float32),
            pltpu.VMEM_SHARED((NACC, DW), jnp.float32),
            pltpu.SemaphoreType.DMA,
        ],
    )
    def k(dst_hbm, ones_hbm, z_hbm, out_hbm, dslab, ones_v, acc, sem):
        cid = lax.axis_index("c")
        sid = lax.axis_index("s")
        wid = sid * NC + cid
        r0 = sid * RPT

        pltpu.sync_copy(z_hbm, acc.at[pl.ds(r0, RPT)])
        pltpu.sync_copy(ones_hbm, ones_v)
        pltpu.sync_copy(dst_hbm.at[wid], dslab)
        plsc.subcore_barrier()

        @pl.loop(0, NCHUNK, step=4)
        def _(j):
            pltpu.async_copy(ones_v, acc.at[dslab.at[j]], sem, add=True)
            pltpu.async_copy(ones_v, acc.at[dslab.at[j + 1]], sem, add=True)
            pltpu.async_copy(ones_v, acc.at[dslab.at[j + 2]], sem, add=True)
            pltpu.async_copy(ones_v, acc.at[dslab.at[j + 3]], sem, add=True)
            pltpu.make_async_copy(ones_v, acc.at[dslab.at[j]], sem).wait()
            pltpu.make_async_copy(ones_v, acc.at[dslab.at[j + 1]], sem).wait()
            pltpu.make_async_copy(ones_v, acc.at[dslab.at[j + 2]], sem).wait()
            pltpu.make_async_copy(ones_v, acc.at[dslab.at[j + 3]], sem).wait()

        plsc.subcore_barrier()
        pltpu.sync_copy(acc.at[pl.ds(r0, RPT)], out_hbm.at[cid, pl.ds(r0, RPT)])

    return k(dstr, ones, zrows)


NP = N // 2
NPA = NACC // 2
BKP = 128
NPADP = NP + BKP

_VM = pl.BlockSpec(memory_space=pltpu.VMEM)


def _fold_mean(m):
    h = (m[:, 0:H] + m[:, H:2 * H]) * 0.5
    return jnp.concatenate([h, h], axis=1)


def _tc_head(xP, w1s, degp128, batch2):
    def body(x_ref, w_ref, d_ref, bt_ref, dinv_ref, hs_ref, co_ref):
        dsum = d_ref[0:1280, :] + d_ref[1280:2560, :]
        r = lax.broadcasted_iota(jnp.int32, (128, 8), 0)
        c = lax.broadcasted_iota(jnp.int32, (128, 8), 1)
        sel = (r == 16 * c).astype(jnp.float32)
        deg8 = lax.dot_general(dsum, sel, (((1,), (0,)), ((), ())),
                               precision=lax.Precision.HIGHEST,
                               preferred_element_type=jnp.float32)
        dinv8 = lax.rsqrt(1.0 + deg8)
        quads = []
        for q in range(4):
            e = jnp.broadcast_to(dinv8[:, 2 * q:2 * q + 1], (1280, H))
            o = jnp.broadcast_to(dinv8[:, 2 * q + 1:2 * q + 2], (1280, H))
            quads.append(jnp.concatenate([e, o], axis=1))
        dinvP = jnp.stack(quads, axis=1).reshape(NPA, 128)
        dinv_ref[...] = dinvP
        z1 = jnp.dot(x_ref[...], w_ref[...],
                     preferred_element_type=jnp.float32)
        hs_ref[...] = z1 * dinvP[0:NP, :]
        bt = bt_ref[...]
        gi = lax.broadcasted_iota(jnp.int32, (N, G), 1)
        co_ref[0:1, :] = jnp.sum((bt == gi).astype(jnp.int32), axis=0,
                                 keepdims=True)
        co_ref[1:2, :] = jnp.sum((bt < gi).astype(jnp.int32), axis=0,
                                 keepdims=True)

    return pl.pallas_call(
        body,
        out_shape=(jax.ShapeDtypeStruct((NPA, 128), jnp.float32),
                   jax.ShapeDtypeStruct((NP, 128), jnp.float32),
                   jax.ShapeDtypeStruct((2, G), jnp.int32)))(
            xP, w1s, degp128, batch2)


def _bn_relu_pair(aggP, gP, beP):
    m = _fold_mean(jnp.mean(aggP, axis=0, keepdims=True))
    c = aggP - m
    v = _fold_mean(jnp.mean(c * c, axis=0, keepdims=True))
    return jnp.maximum(c * lax.rsqrt(v + 1e-5) * gP + beP, 0.0)


def _tc_layer(ap128, hsP, dinvP, bP, gP, beP, wns):
    def body(p_ref, hs_ref, dinv_ref, b_ref, g_ref, be_ref, w_ref, o_ref):
        dv = dinv_ref[0:NP, :]
        aggP = dv * (p_ref[0:NP, :] + p_ref[NPA:NPA + NP, :] + hs_ref[...]) \
            + b_ref[...]
        hn = _bn_relu_pair(aggP, g_ref[...], be_ref[...])
        z = jnp.dot(hn, w_ref[...], preferred_element_type=jnp.float32)
        o_ref[...] = z * dv

    return pl.pallas_call(
        body, out_shape=jax.ShapeDtypeStruct((NP, 128), jnp.float32))(
            ap128, hsP, dinvP, bP, gP, beP, wns)


def _tc_final(ap128, hsP, dinvP, bP, gP, beP, co, wo, bo):
    def body(p_ref, hs_ref, dinv_ref, b_ref, g_ref, be_ref, co_ref, wo_ref,
             bo_ref, o_ref, h3_s, mean_s, max_s):
        dv = dinv_ref[0:NP, :]
        aggP = dv * (p_ref[0:NP, :] + p_ref[NPA:NPA + NP, :] + hs_ref[...]) \
            + b_ref[...]
        h3_s[0:NP, :] = _bn_relu_pair(aggP, g_ref[...], be_ref[...])
        h3_s[NP:NPADP, :] = jnp.zeros((NPADP - NP, 128), jnp.float32)

        def seg(gidx, carry):
            cnt = co_ref[0, gidx]
            off = co_ref[1, gidx]
            pr0 = off // 2
            prn = (off + cnt + 1) // 2 - pr0
            nblk = (prn + (BKP - 1)) // BKP

            def blk(i, sm):
                s, mx = sm
                rows = h3_s[pl.ds(pr0 + i * BKP, BKP), :]
                ne = 2 * (lax.broadcasted_iota(jnp.int32, (BKP, 1), 0)
                          + pr0 + i * BKP)
                ve = ((ne >= off) & (ne < off + cnt)).astype(jnp.float32)
                vo = ((ne + 1 >= off) & (ne + 1 < off + cnt)).astype(
                    jnp.float32)
                mask = jnp.concatenate(
                    [jnp.broadcast_to(ve, (BKP, H)),
                     jnp.broadcast_to(vo, (BKP, H))], axis=1) > 0.5
                s = s + jnp.sum(jnp.where(mask, rows, 0.0), axis=0,
                                keepdims=True)
                mx = jnp.maximum(mx, jnp.max(
                    jnp.where(mask, rows, -jnp.inf), axis=0, keepdims=True))
                return s, mx

            s0 = jnp.zeros((1, 128), jnp.float32)
            m0 = jnp.full((1, 128), -jnp.inf, jnp.float32)
            s, mx = lax.fori_loop(0, nblk, blk, (s0, m0))
            s64 = s[:, 0:H] + s[:, H:2 * H]
            m64 = jnp.maximum(mx[:, 0:H], mx[:, H:2 * H])
            cntf = jnp.maximum(cnt, 1).astype(jnp.float32)
            mean_s[pl.ds(gidx, 1), :] = s64 / cntf
            max_s[pl.ds(gidx, 1), :] = jnp.where(cnt > 0, m64, 0.0)
            return carry

        lax.fori_loop(0, G, seg, 0)
        o_ref[...] = (
            jnp.dot(mean_s[...], wo_ref[0:H, :],
                    preferred_element_type=jnp.float32)
            + jnp.dot(max_s[...], wo_ref[H:2 * H, :],
                      preferred_element_type=jnp.float32)
            + bo_ref[...])

    return pl.pallas_call(
        body,
        out_shape=jax.ShapeDtypeStruct((G, 1), jnp.float32),
        in_specs=[_VM, _VM, _VM, _VM, _VM, _VM,
                  pl.BlockSpec(memory_space=pltpu.SMEM), _VM, _VM],
        scratch_shapes=[
            pltpu.VMEM((NPADP, 128), jnp.float32),
            pltpu.VMEM((G, H), jnp.float32),
            pltpu.VMEM((G, H), jnp.float32),
        ])(ap128, hsP, dinvP, bP, gP, beP, co, wo, bo)


def _blockdiag(w, k):
    z = jnp.zeros(w.shape, jnp.float32)
    top = jnp.concatenate([w, z], axis=1)
    bot = jnp.concatenate([z, w], axis=1)
    return jnp.concatenate([top, bot], axis=0)


def kernel(x, edge_index, batch, W1, b1, W2, b2, W3, b3,
           g1, be1, g2, be2, g3, be3, Wo, bo):
    import numpy as _np
    npad_e = E_PAD - E
    pad_i = _np.arange(npad_e, dtype=_np.int32)
    pad_src = jnp.asarray(pad_i % N)
    pad_dst = jnp.asarray(N + pad_i % (NACC - N))
    srcr = jnp.concatenate([edge_index[0], pad_src]).reshape(NW, NCHUNK, CH)
    dstr = jnp.concatenate([edge_index[1], pad_dst]).reshape(NW, NCHUNK, CH)
    batch2 = batch.reshape(N, 1)
    xP = x.reshape(NP, 2 * F_IN)
    w1s = _blockdiag(W1, 2)
    w2s = _blockdiag(W2, 2)
    w3s = _blockdiag(W3, 2)
    bP = [jnp.concatenate([v, v]).reshape(1, 128) for v in
          (b1, g1, be1, b2, g2, be2, b3, g3, be3)]
    bor = bo.reshape(1, 1)
    zH = jnp.zeros((RPT, H), jnp.float32)
    zD = jnp.zeros((RPT, DW), jnp.float32)
    onesD = jnp.ones((CH, DW), jnp.float32)

    degp = _sc_deg(dstr, onesD, zD)
    degp128 = degp.reshape(NC * NACC * DW // 128, 128)
    dinvP, hs1P, co = _tc_head(xP, w1s, degp128, batch2)

    a1 = _sc_agg(hs1P.reshape(N, H), srcr, dstr, zH)
    hs2P = _tc_layer(a1.reshape(NC * NPA, 128), hs1P, dinvP,
                     bP[0], bP[1], bP[2], w2s)
    a2 = _sc_agg(hs2P.reshape(N, H), srcr, dstr, zH)
    hs3P = _tc_layer(a2.reshape(NC * NPA, 128), hs2P, dinvP,
                     bP[3], bP[4], bP[5], w3s)
    a3 = _sc_agg(hs3P.reshape(N, H), srcr, dstr, zH)

    return _tc_final(a3.reshape(NC * NPA, 128), hs3P, dinvP,
                     bP[6], bP[7], bP[8], co, Wo, bor)
